# baseline (speedup 1.0000x reference)
import functools

import jax
import jax.numpy as jnp
from jax import lax
from jax.experimental import pallas as pl
from jax.experimental.pallas import tpu as pltpu
from jax.experimental.pallas import tpu_sc as plsc

DIM = 1024
D2 = DIM // 2
NTOK = 4 * 8192


_IDX_R, _IDX_C = 256, 128


def _table_index_body(tt_ref, tv_ref, np_ref, emb_ref, primT_ref, id_ref,
                      bigT_ref, g_ref):
    id8 = jnp.broadcast_to(id_ref[...], (8, D2))
    p0t = primT_ref[0]
    p1t = primT_ref[1]
    x1 = jnp.dot(id8, p0t, preferred_element_type=jnp.float32)
    y1 = jnp.dot(id8, p1t, preferred_element_type=jnp.float32)
    x2 = jnp.dot(x1, p0t, preferred_element_type=jnp.float32)
    y2 = jnp.dot(y1, p0t, preferred_element_type=jnp.float32)
    rid = lax.broadcasted_iota(jnp.int32, (8, D2), 0)
    p8 = jnp.where(rid < 2, id8,
         jnp.where(rid == 2, x1,
         jnp.where(rid == 3, y1,
         jnp.where(rid == 4, x2,
         jnp.where(rid == 5, y2, jnp.zeros_like(id8))))))

    bigT_ref[:, :, D2:] = jnp.broadcast_to(p8[None, :, :], (24, 8, D2))
    bigT_ref[0:8, :, 0:D2] = jnp.broadcast_to(p8[:, None, :], (8, 8, D2))
    bigT_ref[8:24, :, 0:D2] = jnp.broadcast_to(emb_ref[...][:, None, :],
                                               (16, 8, D2))

    tt = tt_ref[...]
    tv = tv_ref[...]
    npos = np_ref[...]
    present = [jnp.sum(jnp.where(npos == v, 1, 0)) > 0 for v in range(6)]
    db = jnp.full((_IDX_R, _IDX_C), -1, jnp.int32)
    for v in range(5, -1, -1):
        db = jnp.where(jnp.logical_and(present[v], tv <= v), v, db)
    mp = jnp.int32(-1)
    for v in range(6):
        mp = jnp.where(present[v], jnp.int32(v), mp)
    db = jnp.where(db >= 0, db, mp)

    c = jnp.full((_IDX_R, _IDX_C), 19, jnp.int32)
    c = jnp.where(tt == 0, 8, c)
    c = jnp.where(tt == 1, 9 + tv, c)
    c = jnp.where(tt == 2, 13 + tv, c)
    c = jnp.where(tt == 4, db, c)
    g_ref[...] = c * 8 + npos


def _build_table_and_index(tt, tv, npos, emb16, primT, identity):
    return pl.pallas_call(
        _table_index_body,
        out_shape=[
            jax.ShapeDtypeStruct((24, 8, DIM), jnp.float32),
            jax.ShapeDtypeStruct((_IDX_R, _IDX_C), jnp.int32),
        ],
    )(tt, tv, npos, emb16, primT, identity)


def _table_index_body_p8(tt_ref, tv_ref, np_ref, emb_ref, p8_ref,
                         bigT_ref, g_ref):
    p8 = p8_ref[...]
    bigT_ref[:, :, D2:] = jnp.broadcast_to(p8[None, :, :], (24, 8, D2))
    bigT_ref[0:8, :, 0:D2] = jnp.broadcast_to(p8[:, None, :], (8, 8, D2))
    bigT_ref[8:24, :, 0:D2] = jnp.broadcast_to(emb_ref[...][:, None, :],
                                               (16, 8, D2))
    tt = tt_ref[...]
    tv = tv_ref[...]
    npos = np_ref[...]
    present = [jnp.sum(jnp.where(npos == v, 1, 0)) > 0 for v in range(6)]
    db = jnp.full((_IDX_R, _IDX_C), -1, jnp.int32)
    for v in range(5, -1, -1):
        db = jnp.where(jnp.logical_and(present[v], tv <= v), v, db)
    mp = jnp.int32(-1)
    for v in range(6):
        mp = jnp.where(present[v], jnp.int32(v), mp)
    db = jnp.where(db >= 0, db, mp)
    c = jnp.full((_IDX_R, _IDX_C), 19, jnp.int32)
    c = jnp.where(tt == 0, 8, c)
    c = jnp.where(tt == 1, 9 + tv, c)
    c = jnp.where(tt == 2, 13 + tv, c)
    c = jnp.where(tt == 4, db, c)
    g_ref[...] = c * 8 + npos


def _build_table_and_index_p8(tt, tv, npos, emb16, p8):
    return pl.pallas_call(
        _table_index_body_p8,
        out_shape=[
            jax.ShapeDtypeStruct((24, 8, DIM), jnp.float32),
            jax.ShapeDtypeStruct((_IDX_R, _IDX_C), jnp.int32),
        ],
    )(tt, tv, npos, emb16, p8)


def _fused_body(tt_ref, tv_ref, np_ref, emb_ref, blkN_ref, vv_ref, v_ref,
                bigT_ref, g_ref):
    _cg_into(blkN_ref, vv_ref, v_ref, bigT_ref, emb_ref)
    _index_into(tt_ref, tv_ref, np_ref, g_ref)


def _cg_into(blkN_ref, vv_ref, v_ref, bigT_ref, emb_ref):
    Nb = blkN_ref[...]
    N0 = blkN_ref[0:D2, 0:D2]
    vv8 = jnp.broadcast_to(vv_ref[...], (8, DIM))
    b1 = vv8 - jnp.dot(vv8, Nb, preferred_element_type=jnp.float32)
    n1 = b1 - jnp.dot(b1, Nb, preferred_element_type=jnp.float32)
    x1 = _cg_rows(Nb, n1, _CG_IT1)
    e2 = x1[0:1, 0:D2]
    e3 = x1[0:1, D2:DIM]
    rid = lax.broadcasted_iota(jnp.int32, (8, D2), 0)
    S = jnp.where(rid % 2 == 0, jnp.broadcast_to(e2, (8, D2)),
                  jnp.broadcast_to(e3, (8, D2)))
    b2 = S - jnp.dot(S, N0, preferred_element_type=jnp.float32)
    n2 = b2 - jnp.dot(b2, N0, preferred_element_type=jnp.float32)
    x2 = _cg_rows(N0, n2, _CG_IT2)
    vb = jnp.broadcast_to(v_ref[...], (8, D2))
    p8 = jnp.where(rid < 2, vb,
         jnp.where(rid == 2, jnp.broadcast_to(e2, (8, D2)),
         jnp.where(rid == 3, jnp.broadcast_to(e3, (8, D2)),
         jnp.where(rid < 6, x2, jnp.zeros((8, D2), jnp.float32)))))
    bigT_ref[:, :, D2:] = jnp.broadcast_to(p8[None, :, :], (24, 8, D2))
    bigT_ref[0:8, :, 0:D2] = jnp.broadcast_to(p8[:, None, :], (8, 8, D2))
    bigT_ref[8:24, :, 0:D2] = jnp.broadcast_to(emb_ref[...][:, None, :],
                                               (16, 8, D2))


def _index_into(tt_ref, tv_ref, np_ref, g_ref):
    tt = tt_ref[...]
    tv = tv_ref[...]
    npos = np_ref[...]
    present = [jnp.sum(jnp.where(npos == v, 1, 0)) > 0 for v in range(6)]
    db = jnp.full((_IDX_R, _IDX_C), -1, jnp.int32)
    for v in range(5, -1, -1):
        db = jnp.where(jnp.logical_and(present[v], tv <= v), v, db)
    mp = jnp.int32(-1)
    for v in range(6):
        mp = jnp.where(present[v], jnp.int32(v), mp)
    db = jnp.where(db >= 0, db, mp)
    c = jnp.full((_IDX_R, _IDX_C), 19, jnp.int32)
    c = jnp.where(tt == 0, 8, c)
    c = jnp.where(tt == 1, 9 + tv, c)
    c = jnp.where(tt == 2, 13 + tv, c)
    c = jnp.where(tt == 4, db, c)
    g_ref[...] = c * 8 + npos


def _fused_table_index(tt, tv, npos, emb16, blkN, vv, v):
    return pl.pallas_call(
        _fused_body,
        out_shape=[
            jax.ShapeDtypeStruct((24, 8, DIM), jnp.float32),
            jax.ShapeDtypeStruct((_IDX_R, _IDX_C), jnp.int32),
        ],
    )(tt, tv, npos, emb16, blkN, vv, v)


_CG_IT1 = 110
_CG_IT2 = 110


def _cg_rows(Nmat, nrhs, iters):
    def matvec(p):
        t = jnp.dot(p, Nmat, preferred_element_type=jnp.float32)
        return p - jnp.dot(t, Nmat, preferred_element_type=jnp.float32)

    x0 = jnp.zeros_like(nrhs)
    rs0 = jnp.sum(nrhs * nrhs, axis=1, keepdims=True)

    def it(_, carry):
        x, r, p, rs = carry
        q = matvec(p)
        alpha = rs / jnp.sum(p * q, axis=1, keepdims=True)
        x = x + alpha * p
        r = r - alpha * q
        rs2 = jnp.sum(r * r, axis=1, keepdims=True)
        p = r + (rs2 / rs) * p
        return x, r, p, rs2

    x, _, _, _ = lax.fori_loop(0, iters, it, (x0, nrhs, nrhs, rs0))
    return x


def _cg_body(blkN_ref, vv_ref, v_ref, p8_ref):
    Nb = blkN_ref[...]
    N0 = blkN_ref[0:D2, 0:D2]
    vv8 = jnp.broadcast_to(vv_ref[...], (8, DIM))

    b1 = vv8 - jnp.dot(vv8, Nb, preferred_element_type=jnp.float32)
    n1 = b1 - jnp.dot(b1, Nb, preferred_element_type=jnp.float32)
    x1 = _cg_rows(Nb, n1, _CG_IT1)
    e2 = x1[0:1, 0:D2]
    e3 = x1[0:1, D2:DIM]

    rid = lax.broadcasted_iota(jnp.int32, (8, D2), 0)
    S = jnp.where(rid % 2 == 0, jnp.broadcast_to(e2, (8, D2)),
                  jnp.broadcast_to(e3, (8, D2)))
    b2 = S - jnp.dot(S, N0, preferred_element_type=jnp.float32)
    n2 = b2 - jnp.dot(b2, N0, preferred_element_type=jnp.float32)
    x2 = _cg_rows(N0, n2, _CG_IT2)

    vb = jnp.broadcast_to(v_ref[...], (8, D2))
    p8_ref[...] = jnp.where(rid < 2, vb,
                  jnp.where(rid == 2, jnp.broadcast_to(e2, (8, D2)),
                  jnp.where(rid == 3, jnp.broadcast_to(e3, (8, D2)),
                  jnp.where(rid < 6, x2, jnp.zeros((8, D2), jnp.float32)))))


def _cg_p8(blkN, vv, v):
    return pl.pallas_call(
        _cg_body,
        out_shape=jax.ShapeDtypeStruct((8, D2), jnp.float32),
    )(blkN, vv, v)


_NC = 2
_NS = 16
_NW = _NC * _NS
_BPW = NTOK // _NW
_K = 64
_NCHUNK = _BPW // _K


def _gather_body(table_hbm, idx_hbm, out_hbm, idx_v, rows_v, sem):
    wid = lax.axis_index("s") * _NC + lax.axis_index("c")
    base = wid * _BPW
    pltpu.sync_copy(idx_hbm.at[wid], idx_v)
    for ck in range(_NCHUNK):
        pltpu.async_copy(table_hbm.at[idx_v.at[ck]], rows_v, sem).wait()
        pltpu.sync_copy(rows_v, out_hbm.at[pl.ds(base + ck * _K, _K)])


@functools.cache
def _gather_rows_kernel():
    return functools.partial(
        pl.kernel,
        mesh=plsc.VectorSubcoreMesh(core_axis_name="c", subcore_axis_name="s"),
        out_type=jax.ShapeDtypeStruct((NTOK, DIM), jnp.float32),
        scratch_types=[
            pltpu.VMEM((_NCHUNK, _K), jnp.int32),
            pltpu.VMEM((_K, DIM), jnp.float32),
            pltpu.SemaphoreType.DMA,
        ],
    )(_gather_body)


_K2 = 32
_NCHUNK2 = _BPW // _K2


def _gather_body_pipe(table_hbm, idx_hbm, out_hbm,
                      idx_v, buf0, buf1, sg0, sg1, sw0, sw1):
    wid = lax.axis_index("s") * _NC + lax.axis_index("c")
    base = wid * _BPW
    pltpu.sync_copy(idx_hbm.at[wid], idx_v)
    bufs, sgs, sws = (buf0, buf1), (sg0, sg1), (sw0, sw1)

    def gather(ck, b):
        pltpu.async_copy(table_hbm.at[idx_v.at[ck]], bufs[b], sgs[b])

    def write(ck, b):
        pltpu.async_copy(bufs[b], out_hbm.at[pl.ds(base + ck * _K2, _K2)],
                         sws[b])

    def wait_w(b):
        pltpu.make_async_copy(bufs[b], out_hbm.at[pl.ds(base, _K2)],
                              sws[b]).wait()

    def wait_g(b):
        pltpu.make_async_copy(table_hbm.at[idx_v.at[0]], bufs[b],
                              sgs[b]).wait()

    def chunk(ck, j, first, last):
        b = j % 2
        nb = (j + 1) % 2
        if not last:
            if not first:
                wait_w(nb)
            gather(ck + 1, nb)
        wait_g(b)
        write(ck, b)

    gather(0, 0)
    for j in range(4):
        chunk(j, j, first=(j == 0), last=False)

    def outer(t, carry):
        for j in range(4):
            chunk(4 * t + j, j, first=False, last=False)
        return carry

    lax.fori_loop(1, _NCHUNK2 // 4 - 1, outer, 0)
    for j in range(4):
        ck = _NCHUNK2 - 4 + j
        chunk(ck, j, first=False, last=(j == 3))
    wait_w(0)
    wait_w(1)


@functools.cache
def _gather_rows_pipe_kernel():
    return functools.partial(
        pl.kernel,
        mesh=plsc.VectorSubcoreMesh(core_axis_name="c", subcore_axis_name="s"),
        out_type=jax.ShapeDtypeStruct((NTOK, DIM), jnp.float32),
        scratch_types=[
            pltpu.VMEM((_NCHUNK2, _K2), jnp.int32),
            pltpu.VMEM((_K2, DIM), jnp.float32),
            pltpu.VMEM((_K2, DIM), jnp.float32),
            pltpu.SemaphoreType.DMA,
            pltpu.SemaphoreType.DMA,
            pltpu.SemaphoreType.DMA,
            pltpu.SemaphoreType.DMA,
        ],
    )(_gather_body_pipe)


def _prepare(dense_batch, embeddings, primitives_raw, identity):
    f32 = jnp.float32
    X = jnp.tril(primitives_raw.astype(f32))
    A = X - jnp.swapaxes(X, -1, -2)
    N = 0.5 * A
    zblk = jnp.zeros((D2, D2), f32)
    blkN = jnp.block([[N[0], zblk], [zblk, N[1]]])

    v = identity.astype(f32).reshape(1, D2)
    vv = jnp.concatenate([v, v], axis=1)

    emb16 = jnp.pad(embeddings.astype(f32), ((0, 5), (0, 0)))
    tt = dense_batch[0].reshape(_IDX_R, _IDX_C)
    tv = dense_batch[1].reshape(_IDX_R, _IDX_C)
    npos = dense_batch[2].reshape(_IDX_R, _IDX_C)
    return tt, tv, npos, emb16, blkN, vv, v


def kernel(dense_batch, embeddings, primitives_raw, identity):
    tt, tv, npos, emb16, blkN, vv, v = _prepare(dense_batch, embeddings,
                                                primitives_raw, identity)
    bigT3, g = _fused_table_index(tt, tv, npos, emb16, blkN, vv, v)
    bigT = bigT3.reshape(24 * 8, DIM)
    gidx = g.reshape(_NW, _NCHUNK2, _K2)

    out = _gather_rows_pipe_kernel()(bigT, gidx)
    return out.reshape(4, 8192, DIM)


# speedup vs baseline: 1.0860x; 1.0860x over previous
import functools

import jax
import jax.numpy as jnp
from jax import lax
from jax.experimental import pallas as pl
from jax.experimental.pallas import tpu as pltpu
from jax.experimental.pallas import tpu_sc as plsc

DIM = 1024
D2 = DIM // 2
NTOK = 4 * 8192


_IDX_R, _IDX_C = 256, 128


def _table_index_body(tt_ref, tv_ref, np_ref, emb_ref, primT_ref, id_ref,
                      bigT_ref, g_ref):
    id8 = jnp.broadcast_to(id_ref[...], (8, D2))
    p0t = primT_ref[0]
    p1t = primT_ref[1]
    x1 = jnp.dot(id8, p0t, preferred_element_type=jnp.float32)
    y1 = jnp.dot(id8, p1t, preferred_element_type=jnp.float32)
    x2 = jnp.dot(x1, p0t, preferred_element_type=jnp.float32)
    y2 = jnp.dot(y1, p0t, preferred_element_type=jnp.float32)
    rid = lax.broadcasted_iota(jnp.int32, (8, D2), 0)
    p8 = jnp.where(rid < 2, id8,
         jnp.where(rid == 2, x1,
         jnp.where(rid == 3, y1,
         jnp.where(rid == 4, x2,
         jnp.where(rid == 5, y2, jnp.zeros_like(id8))))))

    bigT_ref[:, :, D2:] = jnp.broadcast_to(p8[None, :, :], (24, 8, D2))
    bigT_ref[0:8, :, 0:D2] = jnp.broadcast_to(p8[:, None, :], (8, 8, D2))
    bigT_ref[8:24, :, 0:D2] = jnp.broadcast_to(emb_ref[...][:, None, :],
                                               (16, 8, D2))

    tt = tt_ref[...]
    tv = tv_ref[...]
    npos = np_ref[...]
    present = [jnp.sum(jnp.where(npos == v, 1, 0)) > 0 for v in range(6)]
    db = jnp.full((_IDX_R, _IDX_C), -1, jnp.int32)
    for v in range(5, -1, -1):
        db = jnp.where(jnp.logical_and(present[v], tv <= v), v, db)
    mp = jnp.int32(-1)
    for v in range(6):
        mp = jnp.where(present[v], jnp.int32(v), mp)
    db = jnp.where(db >= 0, db, mp)

    c = jnp.full((_IDX_R, _IDX_C), 19, jnp.int32)
    c = jnp.where(tt == 0, 8, c)
    c = jnp.where(tt == 1, 9 + tv, c)
    c = jnp.where(tt == 2, 13 + tv, c)
    c = jnp.where(tt == 4, db, c)
    g_ref[...] = c * 8 + npos


def _build_table_and_index(tt, tv, npos, emb16, primT, identity):
    return pl.pallas_call(
        _table_index_body,
        out_shape=[
            jax.ShapeDtypeStruct((24, 8, DIM), jnp.float32),
            jax.ShapeDtypeStruct((_IDX_R, _IDX_C), jnp.int32),
        ],
    )(tt, tv, npos, emb16, primT, identity)


def _table_index_body_p8(tt_ref, tv_ref, np_ref, emb_ref, p8_ref,
                         bigT_ref, g_ref):
    p8 = p8_ref[...]
    bigT_ref[:, :, D2:] = jnp.broadcast_to(p8[None, :, :], (24, 8, D2))
    bigT_ref[0:8, :, 0:D2] = jnp.broadcast_to(p8[:, None, :], (8, 8, D2))
    bigT_ref[8:24, :, 0:D2] = jnp.broadcast_to(emb_ref[...][:, None, :],
                                               (16, 8, D2))
    tt = tt_ref[...]
    tv = tv_ref[...]
    npos = np_ref[...]
    present = [jnp.sum(jnp.where(npos == v, 1, 0)) > 0 for v in range(6)]
    db = jnp.full((_IDX_R, _IDX_C), -1, jnp.int32)
    for v in range(5, -1, -1):
        db = jnp.where(jnp.logical_and(present[v], tv <= v), v, db)
    mp = jnp.int32(-1)
    for v in range(6):
        mp = jnp.where(present[v], jnp.int32(v), mp)
    db = jnp.where(db >= 0, db, mp)
    c = jnp.full((_IDX_R, _IDX_C), 19, jnp.int32)
    c = jnp.where(tt == 0, 8, c)
    c = jnp.where(tt == 1, 9 + tv, c)
    c = jnp.where(tt == 2, 13 + tv, c)
    c = jnp.where(tt == 4, db, c)
    g_ref[...] = c * 8 + npos


def _build_table_and_index_p8(tt, tv, npos, emb16, p8):
    return pl.pallas_call(
        _table_index_body_p8,
        out_shape=[
            jax.ShapeDtypeStruct((24, 8, DIM), jnp.float32),
            jax.ShapeDtypeStruct((_IDX_R, _IDX_C), jnp.int32),
        ],
    )(tt, tv, npos, emb16, p8)


def _fused_body(tt_ref, tv_ref, np_ref, emb_ref, N0_ref, N1_ref, v_ref,
                bigT_ref, g_ref):
    _cg_into(N0_ref, N1_ref, v_ref, bigT_ref, emb_ref)
    _index_into(tt_ref, tv_ref, np_ref, g_ref)


def _cg_rows2(N0, N1, na, nb, iters):
    def mv(p, Nm):
        t = jnp.dot(p, Nm, preferred_element_type=jnp.float32)
        return p - jnp.dot(t, Nm, preferred_element_type=jnp.float32)

    def rdot(a, b):
        return jnp.sum(a * b, axis=1, keepdims=True)

    def it(_, carry):
        xa, ra, pa, rsa, xb, rb, pb, rsb = carry
        qa = mv(pa, N0)
        qb = mv(pb, N1)
        aa = rsa / rdot(pa, qa)
        ab = rsb / rdot(pb, qb)
        xa = xa + aa * pa
        xb = xb + ab * pb
        ra = ra - aa * qa
        rb = rb - ab * qb
        rsa2 = rdot(ra, ra)
        rsb2 = rdot(rb, rb)
        pa = ra + (rsa2 / rsa) * pa
        pb = rb + (rsb2 / rsb) * pb
        return xa, ra, pa, rsa2, xb, rb, pb, rsb2

    z = jnp.zeros_like(na)
    carry = (z, na, na, rdot(na, na), z, nb, nb, rdot(nb, nb))
    out = lax.fori_loop(0, iters, it, carry)
    return out[0], out[4]


def _cg_into(N0_ref, N1_ref, v_ref, bigT_ref, emb_ref):
    N0 = N0_ref[...]
    N1 = N1_ref[...]
    v8 = jnp.broadcast_to(v_ref[...], (8, D2))

    def nrhs(S, Nm):
        b = S - jnp.dot(S, Nm, preferred_element_type=jnp.float32)
        return b - jnp.dot(b, Nm, preferred_element_type=jnp.float32)

    xa, xb = _cg_rows2(N0, N1, nrhs(v8, N0), nrhs(v8, N1), _CG_IT1)
    e2 = xa[0:1, :]
    e3 = xb[0:1, :]

    rid = lax.broadcasted_iota(jnp.int32, (8, D2), 0)
    S = jnp.where(rid % 2 == 0, jnp.broadcast_to(e2, (8, D2)),
                  jnp.broadcast_to(e3, (8, D2)))
    x2 = _cg_rows(N0, nrhs(S, N0), _CG_IT2)

    vb = jnp.broadcast_to(v_ref[...], (8, D2))
    p8 = jnp.where(rid < 2, vb,
         jnp.where(rid == 2, jnp.broadcast_to(e2, (8, D2)),
         jnp.where(rid == 3, jnp.broadcast_to(e3, (8, D2)),
         jnp.where(rid < 6, x2, jnp.zeros((8, D2), jnp.float32)))))
    bigT_ref[:, :, D2:] = jnp.broadcast_to(p8[None, :, :], (24, 8, D2))
    bigT_ref[0:8, :, 0:D2] = jnp.broadcast_to(p8[:, None, :], (8, 8, D2))
    bigT_ref[8:24, :, 0:D2] = jnp.broadcast_to(emb_ref[...][:, None, :],
                                               (16, 8, D2))


def _index_into(tt_ref, tv_ref, np_ref, g_ref):
    tt = tt_ref[...]
    tv = tv_ref[...]
    npos = np_ref[...]
    present = [jnp.sum(jnp.where(npos == v, 1, 0)) > 0 for v in range(6)]
    db = jnp.full((_IDX_R, _IDX_C), -1, jnp.int32)
    for v in range(5, -1, -1):
        db = jnp.where(jnp.logical_and(present[v], tv <= v), v, db)
    mp = jnp.int32(-1)
    for v in range(6):
        mp = jnp.where(present[v], jnp.int32(v), mp)
    db = jnp.where(db >= 0, db, mp)
    c = jnp.full((_IDX_R, _IDX_C), 19, jnp.int32)
    c = jnp.where(tt == 0, 8, c)
    c = jnp.where(tt == 1, 9 + tv, c)
    c = jnp.where(tt == 2, 13 + tv, c)
    c = jnp.where(tt == 4, db, c)
    g_ref[...] = c * 8 + npos


def _fused_table_index(tt, tv, npos, emb16, N0, N1, v):
    return pl.pallas_call(
        _fused_body,
        out_shape=[
            jax.ShapeDtypeStruct((24, 8, DIM), jnp.float32),
            jax.ShapeDtypeStruct((_IDX_R, _IDX_C), jnp.int32),
        ],
    )(tt, tv, npos, emb16, N0, N1, v)


_CG_IT1 = 110
_CG_IT2 = 110


def _cg_rows(Nmat, nrhs, iters):
    def matvec(p):
        t = jnp.dot(p, Nmat, preferred_element_type=jnp.float32)
        return p - jnp.dot(t, Nmat, preferred_element_type=jnp.float32)

    x0 = jnp.zeros_like(nrhs)
    rs0 = jnp.sum(nrhs * nrhs, axis=1, keepdims=True)

    def it(_, carry):
        x, r, p, rs = carry
        q = matvec(p)
        alpha = rs / jnp.sum(p * q, axis=1, keepdims=True)
        x = x + alpha * p
        r = r - alpha * q
        rs2 = jnp.sum(r * r, axis=1, keepdims=True)
        p = r + (rs2 / rs) * p
        return x, r, p, rs2

    x, _, _, _ = lax.fori_loop(0, iters, it, (x0, nrhs, nrhs, rs0))
    return x


def _cg_body(blkN_ref, vv_ref, v_ref, p8_ref):
    Nb = blkN_ref[...]
    N0 = blkN_ref[0:D2, 0:D2]
    vv8 = jnp.broadcast_to(vv_ref[...], (8, DIM))

    b1 = vv8 - jnp.dot(vv8, Nb, preferred_element_type=jnp.float32)
    n1 = b1 - jnp.dot(b1, Nb, preferred_element_type=jnp.float32)
    x1 = _cg_rows(Nb, n1, _CG_IT1)
    e2 = x1[0:1, 0:D2]
    e3 = x1[0:1, D2:DIM]

    rid = lax.broadcasted_iota(jnp.int32, (8, D2), 0)
    S = jnp.where(rid % 2 == 0, jnp.broadcast_to(e2, (8, D2)),
                  jnp.broadcast_to(e3, (8, D2)))
    b2 = S - jnp.dot(S, N0, preferred_element_type=jnp.float32)
    n2 = b2 - jnp.dot(b2, N0, preferred_element_type=jnp.float32)
    x2 = _cg_rows(N0, n2, _CG_IT2)

    vb = jnp.broadcast_to(v_ref[...], (8, D2))
    p8_ref[...] = jnp.where(rid < 2, vb,
                  jnp.where(rid == 2, jnp.broadcast_to(e2, (8, D2)),
                  jnp.where(rid == 3, jnp.broadcast_to(e3, (8, D2)),
                  jnp.where(rid < 6, x2, jnp.zeros((8, D2), jnp.float32)))))


def _cg_p8(blkN, vv, v):
    return pl.pallas_call(
        _cg_body,
        out_shape=jax.ShapeDtypeStruct((8, D2), jnp.float32),
    )(blkN, vv, v)


_NC = 2
_NS = 16
_NW = _NC * _NS
_BPW = NTOK // _NW
_K = 64
_NCHUNK = _BPW // _K


def _gather_body(table_hbm, idx_hbm, out_hbm, idx_v, rows_v, sem):
    wid = lax.axis_index("s") * _NC + lax.axis_index("c")
    base = wid * _BPW
    pltpu.sync_copy(idx_hbm.at[wid], idx_v)
    for ck in range(_NCHUNK):
        pltpu.async_copy(table_hbm.at[idx_v.at[ck]], rows_v, sem).wait()
        pltpu.sync_copy(rows_v, out_hbm.at[pl.ds(base + ck * _K, _K)])


@functools.cache
def _gather_rows_kernel():
    return functools.partial(
        pl.kernel,
        mesh=plsc.VectorSubcoreMesh(core_axis_name="c", subcore_axis_name="s"),
        out_type=jax.ShapeDtypeStruct((NTOK, DIM), jnp.float32),
        scratch_types=[
            pltpu.VMEM((_NCHUNK, _K), jnp.int32),
            pltpu.VMEM((_K, DIM), jnp.float32),
            pltpu.SemaphoreType.DMA,
        ],
    )(_gather_body)


_K2 = 32
_NCHUNK2 = _BPW // _K2


def _gather_body_pipe(table_hbm, idx_hbm, out_hbm,
                      idx_v, buf0, buf1, sg0, sg1, sw0, sw1):
    wid = lax.axis_index("s") * _NC + lax.axis_index("c")
    base = wid * _BPW
    pltpu.sync_copy(idx_hbm.at[wid], idx_v)
    bufs, sgs, sws = (buf0, buf1), (sg0, sg1), (sw0, sw1)

    def gather(ck, b):
        pltpu.async_copy(table_hbm.at[idx_v.at[ck]], bufs[b], sgs[b])

    def write(ck, b):
        pltpu.async_copy(bufs[b], out_hbm.at[pl.ds(base + ck * _K2, _K2)],
                         sws[b])

    def wait_w(b):
        pltpu.make_async_copy(bufs[b], out_hbm.at[pl.ds(base, _K2)],
                              sws[b]).wait()

    def wait_g(b):
        pltpu.make_async_copy(table_hbm.at[idx_v.at[0]], bufs[b],
                              sgs[b]).wait()

    def chunk(ck, j, first, last):
        b = j % 2
        nb = (j + 1) % 2
        if not last:
            if not first:
                wait_w(nb)
            gather(ck + 1, nb)
        wait_g(b)
        write(ck, b)

    gather(0, 0)
    for j in range(4):
        chunk(j, j, first=(j == 0), last=False)

    def outer(t, carry):
        for j in range(4):
            chunk(4 * t + j, j, first=False, last=False)
        return carry

    lax.fori_loop(1, _NCHUNK2 // 4 - 1, outer, 0)
    for j in range(4):
        ck = _NCHUNK2 - 4 + j
        chunk(ck, j, first=False, last=(j == 3))
    wait_w(0)
    wait_w(1)


@functools.cache
def _gather_rows_pipe_kernel():
    return functools.partial(
        pl.kernel,
        mesh=plsc.VectorSubcoreMesh(core_axis_name="c", subcore_axis_name="s"),
        out_type=jax.ShapeDtypeStruct((NTOK, DIM), jnp.float32),
        scratch_types=[
            pltpu.VMEM((_NCHUNK2, _K2), jnp.int32),
            pltpu.VMEM((_K2, DIM), jnp.float32),
            pltpu.VMEM((_K2, DIM), jnp.float32),
            pltpu.SemaphoreType.DMA,
            pltpu.SemaphoreType.DMA,
            pltpu.SemaphoreType.DMA,
            pltpu.SemaphoreType.DMA,
        ],
    )(_gather_body_pipe)


def _prepare(dense_batch, embeddings, primitives_raw, identity):
    f32 = jnp.float32
    X = jnp.tril(primitives_raw.astype(f32))
    A = X - jnp.swapaxes(X, -1, -2)
    N = 0.5 * A

    v = identity.astype(f32).reshape(1, D2)

    emb16 = jnp.pad(embeddings.astype(f32), ((0, 5), (0, 0)))
    tt = dense_batch[0].reshape(_IDX_R, _IDX_C)
    tv = dense_batch[1].reshape(_IDX_R, _IDX_C)
    npos = dense_batch[2].reshape(_IDX_R, _IDX_C)
    return tt, tv, npos, emb16, N[0], N[1], v


def kernel(dense_batch, embeddings, primitives_raw, identity):
    tt, tv, npos, emb16, N0, N1, v = _prepare(dense_batch, embeddings,
                                              primitives_raw, identity)
    bigT3, g = _fused_table_index(tt, tv, npos, emb16, N0, N1, v)
    bigT = bigT3.reshape(24 * 8, DIM)
    gidx = g.reshape(_NW, _NCHUNK2, _K2)

    out = _gather_rows_pipe_kernel()(bigT, gidx)
    return out.reshape(4, 8192, DIM)


# speedup vs baseline: 1.2101x; 1.1143x over previous
import functools

import jax
import jax.numpy as jnp
from jax import lax
from jax.experimental import pallas as pl
from jax.experimental.pallas import tpu as pltpu
from jax.experimental.pallas import tpu_sc as plsc

DIM = 1024
D2 = DIM // 2
NTOK = 4 * 8192


_IDX_R, _IDX_C = 256, 128


def _table_index_body(tt_ref, tv_ref, np_ref, emb_ref, primT_ref, id_ref,
                      bigT_ref, g_ref):
    id8 = jnp.broadcast_to(id_ref[...], (8, D2))
    p0t = primT_ref[0]
    p1t = primT_ref[1]
    x1 = jnp.dot(id8, p0t, preferred_element_type=jnp.float32)
    y1 = jnp.dot(id8, p1t, preferred_element_type=jnp.float32)
    x2 = jnp.dot(x1, p0t, preferred_element_type=jnp.float32)
    y2 = jnp.dot(y1, p0t, preferred_element_type=jnp.float32)
    rid = lax.broadcasted_iota(jnp.int32, (8, D2), 0)
    p8 = jnp.where(rid < 2, id8,
         jnp.where(rid == 2, x1,
         jnp.where(rid == 3, y1,
         jnp.where(rid == 4, x2,
         jnp.where(rid == 5, y2, jnp.zeros_like(id8))))))

    bigT_ref[:, :, D2:] = jnp.broadcast_to(p8[None, :, :], (24, 8, D2))
    bigT_ref[0:8, :, 0:D2] = jnp.broadcast_to(p8[:, None, :], (8, 8, D2))
    bigT_ref[8:24, :, 0:D2] = jnp.broadcast_to(emb_ref[...][:, None, :],
                                               (16, 8, D2))

    tt = tt_ref[...]
    tv = tv_ref[...]
    npos = np_ref[...]
    present = [jnp.sum(jnp.where(npos == v, 1, 0)) > 0 for v in range(6)]
    db = jnp.full((_IDX_R, _IDX_C), -1, jnp.int32)
    for v in range(5, -1, -1):
        db = jnp.where(jnp.logical_and(present[v], tv <= v), v, db)
    mp = jnp.int32(-1)
    for v in range(6):
        mp = jnp.where(present[v], jnp.int32(v), mp)
    db = jnp.where(db >= 0, db, mp)

    c = jnp.full((_IDX_R, _IDX_C), 19, jnp.int32)
    c = jnp.where(tt == 0, 8, c)
    c = jnp.where(tt == 1, 9 + tv, c)
    c = jnp.where(tt == 2, 13 + tv, c)
    c = jnp.where(tt == 4, db, c)
    g_ref[...] = c * 8 + npos


def _build_table_and_index(tt, tv, npos, emb16, primT, identity):
    return pl.pallas_call(
        _table_index_body,
        out_shape=[
            jax.ShapeDtypeStruct((24, 8, DIM), jnp.float32),
            jax.ShapeDtypeStruct((_IDX_R, _IDX_C), jnp.int32),
        ],
    )(tt, tv, npos, emb16, primT, identity)


def _table_index_body_p8(tt_ref, tv_ref, np_ref, emb_ref, p8_ref,
                         bigT_ref, g_ref):
    p8 = p8_ref[...]
    bigT_ref[:, :, D2:] = jnp.broadcast_to(p8[None, :, :], (24, 8, D2))
    bigT_ref[0:8, :, 0:D2] = jnp.broadcast_to(p8[:, None, :], (8, 8, D2))
    bigT_ref[8:24, :, 0:D2] = jnp.broadcast_to(emb_ref[...][:, None, :],
                                               (16, 8, D2))
    tt = tt_ref[...]
    tv = tv_ref[...]
    npos = np_ref[...]
    present = [jnp.sum(jnp.where(npos == v, 1, 0)) > 0 for v in range(6)]
    db = jnp.full((_IDX_R, _IDX_C), -1, jnp.int32)
    for v in range(5, -1, -1):
        db = jnp.where(jnp.logical_and(present[v], tv <= v), v, db)
    mp = jnp.int32(-1)
    for v in range(6):
        mp = jnp.where(present[v], jnp.int32(v), mp)
    db = jnp.where(db >= 0, db, mp)
    c = jnp.full((_IDX_R, _IDX_C), 19, jnp.int32)
    c = jnp.where(tt == 0, 8, c)
    c = jnp.where(tt == 1, 9 + tv, c)
    c = jnp.where(tt == 2, 13 + tv, c)
    c = jnp.where(tt == 4, db, c)
    g_ref[...] = c * 8 + npos


def _build_table_and_index_p8(tt, tv, npos, emb16, p8):
    return pl.pallas_call(
        _table_index_body_p8,
        out_shape=[
            jax.ShapeDtypeStruct((24, 8, DIM), jnp.float32),
            jax.ShapeDtypeStruct((_IDX_R, _IDX_C), jnp.int32),
        ],
    )(tt, tv, npos, emb16, p8)


def _fused_body(tt_ref, tv_ref, np_ref, emb_ref, N0_ref, N1_ref, v_ref,
                bigT_ref, g_ref):
    _cg_into(N0_ref, N1_ref, v_ref, bigT_ref, emb_ref)
    _index_into(tt_ref, tv_ref, np_ref, g_ref)


def _cg_rows2(N0, N1, na, nb, iters):
    def mv(p, Nm):
        t = jnp.dot(p, Nm, preferred_element_type=jnp.float32)
        return p - jnp.dot(t, Nm, preferred_element_type=jnp.float32)

    def rdot(a, b):
        return jnp.sum(a * b, axis=1, keepdims=True)

    def it(_, carry):
        xa, ra, pa, rsa, xb, rb, pb, rsb = carry
        qa = mv(pa, N0)
        qb = mv(pb, N1)
        aa = rsa / rdot(pa, qa)
        ab = rsb / rdot(pb, qb)
        xa = xa + aa * pa
        xb = xb + ab * pb
        ra = ra - aa * qa
        rb = rb - ab * qb
        rsa2 = rdot(ra, ra)
        rsb2 = rdot(rb, rb)
        pa = ra + (rsa2 / rsa) * pa
        pb = rb + (rsb2 / rsb) * pb
        return xa, ra, pa, rsa2, xb, rb, pb, rsb2

    z = jnp.zeros_like(na)
    carry = (z, na, na, rdot(na, na), z, nb, nb, rdot(nb, nb))
    out = lax.fori_loop(0, iters, it, carry, unroll=2)
    return out[0], out[4]


def _cg_into(N0_ref, N1_ref, v_ref, bigT_ref, emb_ref):
    N0 = N0_ref[...]
    N1 = N1_ref[...]
    v8 = jnp.broadcast_to(v_ref[...], (8, D2))

    def nrhs(S, Nm):
        b = S - jnp.dot(S, Nm, preferred_element_type=jnp.float32)
        return b - jnp.dot(b, Nm, preferred_element_type=jnp.float32)

    xa, xb = _cg_rows2(N0, N1, nrhs(v8, N0), nrhs(v8, N1), _CG_IT1)
    e2 = xa[0:1, :]
    e3 = xb[0:1, :]

    rid = lax.broadcasted_iota(jnp.int32, (8, D2), 0)
    S = jnp.where(rid % 2 == 0, jnp.broadcast_to(e2, (8, D2)),
                  jnp.broadcast_to(e3, (8, D2)))
    x2 = _cg_rows(N0, nrhs(S, N0), _CG_IT2)

    vb = jnp.broadcast_to(v_ref[...], (8, D2))
    p8 = jnp.where(rid < 2, vb,
         jnp.where(rid == 2, jnp.broadcast_to(e2, (8, D2)),
         jnp.where(rid == 3, jnp.broadcast_to(e3, (8, D2)),
         jnp.where(rid < 6, x2, jnp.zeros((8, D2), jnp.float32)))))
    bigT_ref[:, :, D2:] = jnp.broadcast_to(p8[None, :, :], (24, 8, D2))
    bigT_ref[0:8, :, 0:D2] = jnp.broadcast_to(p8[:, None, :], (8, 8, D2))
    bigT_ref[8:24, :, 0:D2] = jnp.broadcast_to(emb_ref[...][:, None, :],
                                               (16, 8, D2))


def _index_into(tt_ref, tv_ref, np_ref, g_ref):
    tt = tt_ref[...]
    tv = tv_ref[...]
    npos = np_ref[...]
    present = [jnp.sum(jnp.where(npos == v, 1, 0)) > 0 for v in range(6)]
    db = jnp.full((_IDX_R, _IDX_C), -1, jnp.int32)
    for v in range(5, -1, -1):
        db = jnp.where(jnp.logical_and(present[v], tv <= v), v, db)
    mp = jnp.int32(-1)
    for v in range(6):
        mp = jnp.where(present[v], jnp.int32(v), mp)
    db = jnp.where(db >= 0, db, mp)
    c = jnp.full((_IDX_R, _IDX_C), 19, jnp.int32)
    c = jnp.where(tt == 0, 8, c)
    c = jnp.where(tt == 1, 9 + tv, c)
    c = jnp.where(tt == 2, 13 + tv, c)
    c = jnp.where(tt == 4, db, c)
    g_ref[...] = c * 8 + npos


def _fused_table_index(tt, tv, npos, emb16, N0, N1, v):
    return pl.pallas_call(
        _fused_body,
        out_shape=[
            jax.ShapeDtypeStruct((24, 8, DIM), jnp.float32),
            jax.ShapeDtypeStruct((_IDX_R, _IDX_C), jnp.int32),
        ],
    )(tt, tv, npos, emb16, N0, N1, v)


_CG_IT1 = 84
_CG_IT2 = 84


def _cg_rows(Nmat, nrhs, iters):
    def matvec(p):
        t = jnp.dot(p, Nmat, preferred_element_type=jnp.float32)
        return p - jnp.dot(t, Nmat, preferred_element_type=jnp.float32)

    x0 = jnp.zeros_like(nrhs)
    rs0 = jnp.sum(nrhs * nrhs, axis=1, keepdims=True)

    def it(_, carry):
        x, r, p, rs = carry
        q = matvec(p)
        alpha = rs / jnp.sum(p * q, axis=1, keepdims=True)
        x = x + alpha * p
        r = r - alpha * q
        rs2 = jnp.sum(r * r, axis=1, keepdims=True)
        p = r + (rs2 / rs) * p
        return x, r, p, rs2

    x, _, _, _ = lax.fori_loop(0, iters, it, (x0, nrhs, nrhs, rs0), unroll=2)
    return x


def _cg_body(blkN_ref, vv_ref, v_ref, p8_ref):
    Nb = blkN_ref[...]
    N0 = blkN_ref[0:D2, 0:D2]
    vv8 = jnp.broadcast_to(vv_ref[...], (8, DIM))

    b1 = vv8 - jnp.dot(vv8, Nb, preferred_element_type=jnp.float32)
    n1 = b1 - jnp.dot(b1, Nb, preferred_element_type=jnp.float32)
    x1 = _cg_rows(Nb, n1, _CG_IT1)
    e2 = x1[0:1, 0:D2]
    e3 = x1[0:1, D2:DIM]

    rid = lax.broadcasted_iota(jnp.int32, (8, D2), 0)
    S = jnp.where(rid % 2 == 0, jnp.broadcast_to(e2, (8, D2)),
                  jnp.broadcast_to(e3, (8, D2)))
    b2 = S - jnp.dot(S, N0, preferred_element_type=jnp.float32)
    n2 = b2 - jnp.dot(b2, N0, preferred_element_type=jnp.float32)
    x2 = _cg_rows(N0, n2, _CG_IT2)

    vb = jnp.broadcast_to(v_ref[...], (8, D2))
    p8_ref[...] = jnp.where(rid < 2, vb,
                  jnp.where(rid == 2, jnp.broadcast_to(e2, (8, D2)),
                  jnp.where(rid == 3, jnp.broadcast_to(e3, (8, D2)),
                  jnp.where(rid < 6, x2, jnp.zeros((8, D2), jnp.float32)))))


def _cg_p8(blkN, vv, v):
    return pl.pallas_call(
        _cg_body,
        out_shape=jax.ShapeDtypeStruct((8, D2), jnp.float32),
    )(blkN, vv, v)


_NC = 2
_NS = 16
_NW = _NC * _NS
_BPW = NTOK // _NW
_K = 64
_NCHUNK = _BPW // _K


def _gather_body(table_hbm, idx_hbm, out_hbm, idx_v, rows_v, sem):
    wid = lax.axis_index("s") * _NC + lax.axis_index("c")
    base = wid * _BPW
    pltpu.sync_copy(idx_hbm.at[wid], idx_v)
    for ck in range(_NCHUNK):
        pltpu.async_copy(table_hbm.at[idx_v.at[ck]], rows_v, sem).wait()
        pltpu.sync_copy(rows_v, out_hbm.at[pl.ds(base + ck * _K, _K)])


@functools.cache
def _gather_rows_kernel():
    return functools.partial(
        pl.kernel,
        mesh=plsc.VectorSubcoreMesh(core_axis_name="c", subcore_axis_name="s"),
        out_type=jax.ShapeDtypeStruct((NTOK, DIM), jnp.float32),
        scratch_types=[
            pltpu.VMEM((_NCHUNK, _K), jnp.int32),
            pltpu.VMEM((_K, DIM), jnp.float32),
            pltpu.SemaphoreType.DMA,
        ],
    )(_gather_body)


_K2 = 32
_NCHUNK2 = _BPW // _K2


def _gather_body_pipe(table_hbm, idx_hbm, out_hbm,
                      idx_v, buf0, buf1, sg0, sg1, sw0, sw1):
    wid = lax.axis_index("s") * _NC + lax.axis_index("c")
    base = wid * _BPW
    pltpu.sync_copy(idx_hbm.at[wid], idx_v)
    bufs, sgs, sws = (buf0, buf1), (sg0, sg1), (sw0, sw1)

    def gather(ck, b):
        pltpu.async_copy(table_hbm.at[idx_v.at[ck]], bufs[b], sgs[b])

    def write(ck, b):
        pltpu.async_copy(bufs[b], out_hbm.at[pl.ds(base + ck * _K2, _K2)],
                         sws[b])

    def wait_w(b):
        pltpu.make_async_copy(bufs[b], out_hbm.at[pl.ds(base, _K2)],
                              sws[b]).wait()

    def wait_g(b):
        pltpu.make_async_copy(table_hbm.at[idx_v.at[0]], bufs[b],
                              sgs[b]).wait()

    def chunk(ck, j, first, last):
        b = j % 2
        nb = (j + 1) % 2
        if not last:
            if not first:
                wait_w(nb)
            gather(ck + 1, nb)
        wait_g(b)
        write(ck, b)

    gather(0, 0)
    for j in range(4):
        chunk(j, j, first=(j == 0), last=False)

    def outer(t, carry):
        for j in range(4):
            chunk(4 * t + j, j, first=False, last=False)
        return carry

    lax.fori_loop(1, _NCHUNK2 // 4 - 1, outer, 0)
    for j in range(4):
        ck = _NCHUNK2 - 4 + j
        chunk(ck, j, first=False, last=(j == 3))
    wait_w(0)
    wait_w(1)


@functools.cache
def _gather_rows_pipe_kernel():
    return functools.partial(
        pl.kernel,
        mesh=plsc.VectorSubcoreMesh(core_axis_name="c", subcore_axis_name="s"),
        out_type=jax.ShapeDtypeStruct((NTOK, DIM), jnp.float32),
        scratch_types=[
            pltpu.VMEM((_NCHUNK2, _K2), jnp.int32),
            pltpu.VMEM((_K2, DIM), jnp.float32),
            pltpu.VMEM((_K2, DIM), jnp.float32),
            pltpu.SemaphoreType.DMA,
            pltpu.SemaphoreType.DMA,
            pltpu.SemaphoreType.DMA,
            pltpu.SemaphoreType.DMA,
        ],
    )(_gather_body_pipe)


def _prepare(dense_batch, embeddings, primitives_raw, identity):
    f32 = jnp.float32
    X = jnp.tril(primitives_raw.astype(f32))
    A = X - jnp.swapaxes(X, -1, -2)
    N = 0.5 * A

    v = identity.astype(f32).reshape(1, D2)

    emb16 = jnp.pad(embeddings.astype(f32), ((0, 5), (0, 0)))
    tt = dense_batch[0].reshape(_IDX_R, _IDX_C)
    tv = dense_batch[1].reshape(_IDX_R, _IDX_C)
    npos = dense_batch[2].reshape(_IDX_R, _IDX_C)
    return tt, tv, npos, emb16, N[0], N[1], v


def kernel(dense_batch, embeddings, primitives_raw, identity):
    tt, tv, npos, emb16, N0, N1, v = _prepare(dense_batch, embeddings,
                                              primitives_raw, identity)
    bigT3, g = _fused_table_index(tt, tv, npos, emb16, N0, N1, v)
    bigT = bigT3.reshape(24 * 8, DIM)
    gidx = g.reshape(_NW, _NCHUNK2, _K2)

    out = _gather_rows_pipe_kernel()(bigT, gidx)
    return out.reshape(4, 8192, DIM)


# speedup vs baseline: 2.4064x; 1.9886x over previous
import functools

import jax
import jax.numpy as jnp
from jax import lax
from jax.experimental import pallas as pl
from jax.experimental.pallas import tpu as pltpu
from jax.experimental.pallas import tpu_sc as plsc

DIM = 1024
D2 = DIM // 2
NTOK = 4 * 8192


_IDX_R, _IDX_C = 256, 128


def _table_index_body(tt_ref, tv_ref, np_ref, emb_ref, primT_ref, id_ref,
                      bigT_ref, g_ref):
    id8 = jnp.broadcast_to(id_ref[...], (8, D2))
    p0t = primT_ref[0]
    p1t = primT_ref[1]
    x1 = jnp.dot(id8, p0t, preferred_element_type=jnp.float32)
    y1 = jnp.dot(id8, p1t, preferred_element_type=jnp.float32)
    x2 = jnp.dot(x1, p0t, preferred_element_type=jnp.float32)
    y2 = jnp.dot(y1, p0t, preferred_element_type=jnp.float32)
    rid = lax.broadcasted_iota(jnp.int32, (8, D2), 0)
    p8 = jnp.where(rid < 2, id8,
         jnp.where(rid == 2, x1,
         jnp.where(rid == 3, y1,
         jnp.where(rid == 4, x2,
         jnp.where(rid == 5, y2, jnp.zeros_like(id8))))))

    bigT_ref[:, :, D2:] = jnp.broadcast_to(p8[None, :, :], (24, 8, D2))
    bigT_ref[0:8, :, 0:D2] = jnp.broadcast_to(p8[:, None, :], (8, 8, D2))
    bigT_ref[8:24, :, 0:D2] = jnp.broadcast_to(emb_ref[...][:, None, :],
                                               (16, 8, D2))

    tt = tt_ref[...]
    tv = tv_ref[...]
    npos = np_ref[...]
    present = [jnp.sum(jnp.where(npos == v, 1, 0)) > 0 for v in range(6)]
    db = jnp.full((_IDX_R, _IDX_C), -1, jnp.int32)
    for v in range(5, -1, -1):
        db = jnp.where(jnp.logical_and(present[v], tv <= v), v, db)
    mp = jnp.int32(-1)
    for v in range(6):
        mp = jnp.where(present[v], jnp.int32(v), mp)
    db = jnp.where(db >= 0, db, mp)

    c = jnp.full((_IDX_R, _IDX_C), 19, jnp.int32)
    c = jnp.where(tt == 0, 8, c)
    c = jnp.where(tt == 1, 9 + tv, c)
    c = jnp.where(tt == 2, 13 + tv, c)
    c = jnp.where(tt == 4, db, c)
    g_ref[...] = c * 8 + npos


def _build_table_and_index(tt, tv, npos, emb16, primT, identity):
    return pl.pallas_call(
        _table_index_body,
        out_shape=[
            jax.ShapeDtypeStruct((24, 8, DIM), jnp.float32),
            jax.ShapeDtypeStruct((_IDX_R, _IDX_C), jnp.int32),
        ],
    )(tt, tv, npos, emb16, primT, identity)


def _table_index_body_p8(tt_ref, tv_ref, np_ref, emb_ref, p8_ref,
                         bigT_ref, g_ref):
    p8 = p8_ref[...]
    bigT_ref[:, :, D2:] = jnp.broadcast_to(p8[None, :, :], (24, 8, D2))
    bigT_ref[0:8, :, 0:D2] = jnp.broadcast_to(p8[:, None, :], (8, 8, D2))
    bigT_ref[8:24, :, 0:D2] = jnp.broadcast_to(emb_ref[...][:, None, :],
                                               (16, 8, D2))
    tt = tt_ref[...]
    tv = tv_ref[...]
    npos = np_ref[...]
    present = [jnp.sum(jnp.where(npos == v, 1, 0)) > 0 for v in range(6)]
    db = jnp.full((_IDX_R, _IDX_C), -1, jnp.int32)
    for v in range(5, -1, -1):
        db = jnp.where(jnp.logical_and(present[v], tv <= v), v, db)
    mp = jnp.int32(-1)
    for v in range(6):
        mp = jnp.where(present[v], jnp.int32(v), mp)
    db = jnp.where(db >= 0, db, mp)
    c = jnp.full((_IDX_R, _IDX_C), 19, jnp.int32)
    c = jnp.where(tt == 0, 8, c)
    c = jnp.where(tt == 1, 9 + tv, c)
    c = jnp.where(tt == 2, 13 + tv, c)
    c = jnp.where(tt == 4, db, c)
    g_ref[...] = c * 8 + npos


def _build_table_and_index_p8(tt, tv, npos, emb16, p8):
    return pl.pallas_call(
        _table_index_body_p8,
        out_shape=[
            jax.ShapeDtypeStruct((24, 8, DIM), jnp.float32),
            jax.ShapeDtypeStruct((_IDX_R, _IDX_C), jnp.int32),
        ],
    )(tt, tv, npos, emb16, p8)


def _fused_body(tt_ref, tv_ref, np_ref, emb_ref, N0_ref, N1_ref, v_ref,
                bigT_ref, g_ref, tabC_ref, tabP_ref):
    p8 = _cg_into(N0_ref, N1_ref, v_ref, bigT_ref, emb_ref)
    tabC_ref[0:8, :] = p8
    tabC_ref[8:24, :] = emb_ref[...]
    tabP_ref[...] = p8
    _index_into(tt_ref, tv_ref, np_ref, g_ref)


def _cg_rows2(N0, N1, na, nb, iters):
    def mv(p, Nm):
        t = jnp.dot(p, Nm, preferred_element_type=jnp.float32)
        return p - jnp.dot(t, Nm, preferred_element_type=jnp.float32)

    def rdot(a, b):
        return jnp.sum(a * b, axis=1, keepdims=True)

    def it(_, carry):
        xa, ra, pa, rsa, xb, rb, pb, rsb = carry
        qa = mv(pa, N0)
        qb = mv(pb, N1)
        aa = rsa / rdot(pa, qa)
        ab = rsb / rdot(pb, qb)
        xa = xa + aa * pa
        xb = xb + ab * pb
        ra = ra - aa * qa
        rb = rb - ab * qb
        rsa2 = rdot(ra, ra)
        rsb2 = rdot(rb, rb)
        pa = ra + (rsa2 / rsa) * pa
        pb = rb + (rsb2 / rsb) * pb
        return xa, ra, pa, rsa2, xb, rb, pb, rsb2

    z = jnp.zeros_like(na)
    carry = (z, na, na, rdot(na, na), z, nb, nb, rdot(nb, nb))
    out = lax.fori_loop(0, iters, it, carry, unroll=2)
    return out[0], out[4]


def _cg_into(N0_ref, N1_ref, v_ref, bigT_ref, emb_ref):
    N0 = N0_ref[...]
    N1 = N1_ref[...]
    v8 = jnp.broadcast_to(v_ref[...], (8, D2))

    def nrhs(S, Nm):
        b = S - jnp.dot(S, Nm, preferred_element_type=jnp.float32)
        return b - jnp.dot(b, Nm, preferred_element_type=jnp.float32)

    xa, xb = _cg_rows2(N0, N1, nrhs(v8, N0), nrhs(v8, N1), _CG_IT1)
    e2 = xa[0:1, :]
    e3 = xb[0:1, :]

    rid = lax.broadcasted_iota(jnp.int32, (8, D2), 0)
    S = jnp.where(rid % 2 == 0, jnp.broadcast_to(e2, (8, D2)),
                  jnp.broadcast_to(e3, (8, D2)))
    x2 = _cg_rows(N0, nrhs(S, N0), _CG_IT2)

    vb = jnp.broadcast_to(v_ref[...], (8, D2))
    p8 = jnp.where(rid < 2, vb,
         jnp.where(rid == 2, jnp.broadcast_to(e2, (8, D2)),
         jnp.where(rid == 3, jnp.broadcast_to(e3, (8, D2)),
         jnp.where(rid < 6, x2, jnp.zeros((8, D2), jnp.float32)))))
    bigT_ref[:, :, D2:] = jnp.broadcast_to(p8[None, :, :], (24, 8, D2))
    bigT_ref[0:8, :, 0:D2] = jnp.broadcast_to(p8[:, None, :], (8, 8, D2))
    bigT_ref[8:24, :, 0:D2] = jnp.broadcast_to(emb_ref[...][:, None, :],
                                               (16, 8, D2))
    return p8


def _index_into(tt_ref, tv_ref, np_ref, g_ref):
    tt = tt_ref[...]
    tv = tv_ref[...]
    npos = np_ref[...]
    present = [jnp.sum(jnp.where(npos == v, 1, 0)) > 0 for v in range(6)]
    db = jnp.full((_IDX_R, _IDX_C), -1, jnp.int32)
    for v in range(5, -1, -1):
        db = jnp.where(jnp.logical_and(present[v], tv <= v), v, db)
    mp = jnp.int32(-1)
    for v in range(6):
        mp = jnp.where(present[v], jnp.int32(v), mp)
    db = jnp.where(db >= 0, db, mp)
    c = jnp.full((_IDX_R, _IDX_C), 19, jnp.int32)
    c = jnp.where(tt == 0, 8, c)
    c = jnp.where(tt == 1, 9 + tv, c)
    c = jnp.where(tt == 2, 13 + tv, c)
    c = jnp.where(tt == 4, db, c)
    g_ref[...] = c * 8 + npos


def _fused_table_index(tt, tv, npos, emb16, N0, N1, v):
    return pl.pallas_call(
        _fused_body,
        out_shape=[
            jax.ShapeDtypeStruct((24, 8, DIM), jnp.float32),
            jax.ShapeDtypeStruct((_IDX_R, _IDX_C), jnp.int32),
            jax.ShapeDtypeStruct((24, D2), jnp.float32),
            jax.ShapeDtypeStruct((8, D2), jnp.float32),
        ],
    )(tt, tv, npos, emb16, N0, N1, v)


_CG_IT1 = 84
_CG_IT2 = 84


def _cg_rows(Nmat, nrhs, iters):
    def matvec(p):
        t = jnp.dot(p, Nmat, preferred_element_type=jnp.float32)
        return p - jnp.dot(t, Nmat, preferred_element_type=jnp.float32)

    x0 = jnp.zeros_like(nrhs)
    rs0 = jnp.sum(nrhs * nrhs, axis=1, keepdims=True)

    def it(_, carry):
        x, r, p, rs = carry
        q = matvec(p)
        alpha = rs / jnp.sum(p * q, axis=1, keepdims=True)
        x = x + alpha * p
        r = r - alpha * q
        rs2 = jnp.sum(r * r, axis=1, keepdims=True)
        p = r + (rs2 / rs) * p
        return x, r, p, rs2

    x, _, _, _ = lax.fori_loop(0, iters, it, (x0, nrhs, nrhs, rs0), unroll=2)
    return x


def _cg_body(blkN_ref, vv_ref, v_ref, p8_ref):
    Nb = blkN_ref[...]
    N0 = blkN_ref[0:D2, 0:D2]
    vv8 = jnp.broadcast_to(vv_ref[...], (8, DIM))

    b1 = vv8 - jnp.dot(vv8, Nb, preferred_element_type=jnp.float32)
    n1 = b1 - jnp.dot(b1, Nb, preferred_element_type=jnp.float32)
    x1 = _cg_rows(Nb, n1, _CG_IT1)
    e2 = x1[0:1, 0:D2]
    e3 = x1[0:1, D2:DIM]

    rid = lax.broadcasted_iota(jnp.int32, (8, D2), 0)
    S = jnp.where(rid % 2 == 0, jnp.broadcast_to(e2, (8, D2)),
                  jnp.broadcast_to(e3, (8, D2)))
    b2 = S - jnp.dot(S, N0, preferred_element_type=jnp.float32)
    n2 = b2 - jnp.dot(b2, N0, preferred_element_type=jnp.float32)
    x2 = _cg_rows(N0, n2, _CG_IT2)

    vb = jnp.broadcast_to(v_ref[...], (8, D2))
    p8_ref[...] = jnp.where(rid < 2, vb,
                  jnp.where(rid == 2, jnp.broadcast_to(e2, (8, D2)),
                  jnp.where(rid == 3, jnp.broadcast_to(e3, (8, D2)),
                  jnp.where(rid < 6, x2, jnp.zeros((8, D2), jnp.float32)))))


def _cg_p8(blkN, vv, v):
    return pl.pallas_call(
        _cg_body,
        out_shape=jax.ShapeDtypeStruct((8, D2), jnp.float32),
    )(blkN, vv, v)


_NC = 2
_NS = 16
_NW = _NC * _NS
_BPW = NTOK // _NW
_K = 64
_NCHUNK = _BPW // _K


def _gather_body(table_hbm, idx_hbm, out_hbm, idx_v, rows_v, sem):
    wid = lax.axis_index("s") * _NC + lax.axis_index("c")
    base = wid * _BPW
    pltpu.sync_copy(idx_hbm.at[wid], idx_v)
    for ck in range(_NCHUNK):
        pltpu.async_copy(table_hbm.at[idx_v.at[ck]], rows_v, sem).wait()
        pltpu.sync_copy(rows_v, out_hbm.at[pl.ds(base + ck * _K, _K)])


@functools.cache
def _gather_rows_kernel():
    return functools.partial(
        pl.kernel,
        mesh=plsc.VectorSubcoreMesh(core_axis_name="c", subcore_axis_name="s"),
        out_type=jax.ShapeDtypeStruct((NTOK, DIM), jnp.float32),
        scratch_types=[
            pltpu.VMEM((_NCHUNK, _K), jnp.int32),
            pltpu.VMEM((_K, DIM), jnp.float32),
            pltpu.SemaphoreType.DMA,
        ],
    )(_gather_body)


_K2 = 32
_NCHUNK2 = _BPW // _K2


def _gather_body_pipe(table_hbm, idx_hbm, out_hbm,
                      idx_v, buf0, buf1, sg0, sg1, sw0, sw1):
    wid = lax.axis_index("s") * _NC + lax.axis_index("c")
    base = wid * _BPW
    pltpu.sync_copy(idx_hbm.at[wid], idx_v)
    bufs, sgs, sws = (buf0, buf1), (sg0, sg1), (sw0, sw1)

    def gather(ck, b):
        pltpu.async_copy(table_hbm.at[idx_v.at[ck]], bufs[b], sgs[b])

    def write(ck, b):
        pltpu.async_copy(bufs[b], out_hbm.at[pl.ds(base + ck * _K2, _K2)],
                         sws[b])

    def wait_w(b):
        pltpu.make_async_copy(bufs[b], out_hbm.at[pl.ds(base, _K2)],
                              sws[b]).wait()

    def wait_g(b):
        pltpu.make_async_copy(table_hbm.at[idx_v.at[0]], bufs[b],
                              sgs[b]).wait()

    def chunk(ck, j, first, last):
        b = j % 2
        nb = (j + 1) % 2
        if not last:
            if not first:
                wait_w(nb)
            gather(ck + 1, nb)
        wait_g(b)
        write(ck, b)

    gather(0, 0)
    for j in range(4):
        chunk(j, j, first=(j == 0), last=False)

    def outer(t, carry):
        for j in range(4):
            chunk(4 * t + j, j, first=False, last=False)
        return carry

    lax.fori_loop(1, _NCHUNK2 // 4 - 1, outer, 0)
    for j in range(4):
        ck = _NCHUNK2 - 4 + j
        chunk(ck, j, first=False, last=(j == 3))
    wait_w(0)
    wait_w(1)


_KL = 32


def _assemble_body(tabC_hbm, tabP_hbm, idx_hbm, out_hbm,
                   tabC_v, tabP_v, g_v, buf0, buf1, sl0, sl1, sw0, sw1):
    sid = lax.axis_index("s")
    wid = sid * _NC + lax.axis_index("c")
    base = wid * _BPW

    @pl.when(sid == 0)
    def _():
        pltpu.sync_copy(tabC_hbm, tabC_v)
        pltpu.sync_copy(tabP_hbm, tabP_v)

    plsc.subcore_barrier()
    pltpu.sync_copy(idx_hbm.at[wid], g_v)
    g_sm = g_v
    bufs, sls, sws = (buf0, buf1), (sl0, sl1), (sw0, sw1)
    nch = _BPW // _KL

    def wait_w(b):
        pltpu.make_async_copy(bufs[b], out_hbm.at[pl.ds(base, _KL)],
                              sws[b]).wait()

    def drain_l(b):
        pltpu.make_async_copy(out_hbm.at[pl.ds(base, _KL)], bufs[b],
                              sls[b]).wait()

    def chunk(ck, b, first):
        if not first:
            wait_w(b)
        for h in range(_KL // 16):
            gvec = g_v[pl.ds(ck * _KL + h * 16, 16)]
            cvec = jax.lax.shift_right_logical(gvec, 3)
            pvec = jax.lax.bitwise_and(gvec, 7)
            for j in range(16):
                row = h * 16 + j
                pltpu.async_copy(tabC_v.at[cvec[j]],
                                 bufs[b].at[row, pl.ds(0, D2)], sls[b])
                pltpu.async_copy(tabP_v.at[pvec[j]],
                                 bufs[b].at[row, pl.ds(D2, D2)], sls[b])
        drain_l(b)
        pltpu.async_copy(bufs[b], out_hbm.at[pl.ds(base + ck * _KL, _KL)],
                         sws[b])

    chunk(0, 0, True)
    chunk(1, 1, True)

    def outer(t, carry):
        chunk(2 * t, 0, False)
        chunk(2 * t + 1, 1, False)
        return carry

    lax.fori_loop(1, nch // 2, outer, 0)
    wait_w(0)
    wait_w(1)


@functools.cache
def _assemble_rows_kernel():
    return functools.partial(
        pl.kernel,
        mesh=plsc.VectorSubcoreMesh(core_axis_name="c", subcore_axis_name="s"),
        out_type=jax.ShapeDtypeStruct((NTOK, DIM), jnp.float32),
        scratch_types=[
            pltpu.VMEM_SHARED((24, D2), jnp.float32),
            pltpu.VMEM_SHARED((8, D2), jnp.float32),
            pltpu.VMEM((_BPW,), jnp.int32),
            pltpu.VMEM((_KL, DIM), jnp.float32),
            pltpu.VMEM((_KL, DIM), jnp.float32),
            pltpu.SemaphoreType.DMA,
            pltpu.SemaphoreType.DMA,
            pltpu.SemaphoreType.DMA,
            pltpu.SemaphoreType.DMA,
        ],
    )(_assemble_body)


@functools.cache
def _gather_rows_pipe_kernel():
    return functools.partial(
        pl.kernel,
        mesh=plsc.VectorSubcoreMesh(core_axis_name="c", subcore_axis_name="s"),
        out_type=jax.ShapeDtypeStruct((NTOK, DIM), jnp.float32),
        scratch_types=[
            pltpu.VMEM((_NCHUNK2, _K2), jnp.int32),
            pltpu.VMEM((_K2, DIM), jnp.float32),
            pltpu.VMEM((_K2, DIM), jnp.float32),
            pltpu.SemaphoreType.DMA,
            pltpu.SemaphoreType.DMA,
            pltpu.SemaphoreType.DMA,
            pltpu.SemaphoreType.DMA,
        ],
    )(_gather_body_pipe)


def _prepare(dense_batch, embeddings, primitives_raw, identity):
    f32 = jnp.float32
    X = jnp.tril(primitives_raw.astype(f32))
    A = X - jnp.swapaxes(X, -1, -2)
    N = 0.5 * A

    v = identity.astype(f32).reshape(1, D2)

    emb16 = jnp.pad(embeddings.astype(f32), ((0, 5), (0, 0)))
    tt = dense_batch[0].reshape(_IDX_R, _IDX_C)
    tv = dense_batch[1].reshape(_IDX_R, _IDX_C)
    npos = dense_batch[2].reshape(_IDX_R, _IDX_C)
    return tt, tv, npos, emb16, N[0], N[1], v


def kernel(dense_batch, embeddings, primitives_raw, identity):
    tt, tv, npos, emb16, N0, N1, v = _prepare(dense_batch, embeddings,
                                              primitives_raw, identity)
    bigT3, g, tabC, tabP = _fused_table_index(tt, tv, npos, emb16, N0, N1, v)
    gidx = g.reshape(_NW, _BPW)

    out = _assemble_rows_kernel()(tabC, tabP, gidx)
    return out.reshape(4, 8192, DIM)


# speedup vs baseline: 2.4134x; 1.0029x over previous
import functools

import jax
import jax.numpy as jnp
from jax import lax
from jax.experimental import pallas as pl
from jax.experimental.pallas import tpu as pltpu
from jax.experimental.pallas import tpu_sc as plsc

DIM = 1024
D2 = DIM // 2
NTOK = 4 * 8192

_IDX_R, _IDX_C = 256, 128

_CG_IT1 = 84
_CG_IT2 = 84

_NC = 2
_NS = 16
_NW = _NC * _NS
_BPW = NTOK // _NW
_KL = 32


def _cg_rows(Nmat, nrhs, iters):
    def matvec(p):
        t = jnp.dot(p, Nmat, preferred_element_type=jnp.float32)
        return p - jnp.dot(t, Nmat, preferred_element_type=jnp.float32)

    def rdot(a, b):
        return jnp.sum(a * b, axis=1, keepdims=True)

    def it(_, carry):
        x, r, p, rs = carry
        q = matvec(p)
        alpha = rs / rdot(p, q)
        x = x + alpha * p
        r = r - alpha * q
        rs2 = rdot(r, r)
        p = r + (rs2 / rs) * p
        return x, r, p, rs2

    x0 = jnp.zeros_like(nrhs)
    x, _, _, _ = lax.fori_loop(0, iters, it, (x0, nrhs, nrhs, rdot(nrhs, nrhs)),
                               unroll=2)
    return x


def _cg_rows2(N0, N1, na, nb, iters):
    def mv(p, Nm):
        t = jnp.dot(p, Nm, preferred_element_type=jnp.float32)
        return p - jnp.dot(t, Nm, preferred_element_type=jnp.float32)

    def rdot(a, b):
        return jnp.sum(a * b, axis=1, keepdims=True)

    def it(_, carry):
        xa, ra, pa, rsa, xb, rb, pb, rsb = carry
        qa = mv(pa, N0)
        qb = mv(pb, N1)
        aa = rsa / rdot(pa, qa)
        ab = rsb / rdot(pb, qb)
        xa = xa + aa * pa
        xb = xb + ab * pb
        ra = ra - aa * qa
        rb = rb - ab * qb
        rsa2 = rdot(ra, ra)
        rsb2 = rdot(rb, rb)
        pa = ra + (rsa2 / rsa) * pa
        pb = rb + (rsb2 / rsb) * pb
        return xa, ra, pa, rsa2, xb, rb, pb, rsb2

    z = jnp.zeros_like(na)
    carry = (z, na, na, rdot(na, na), z, nb, nb, rdot(nb, nb))
    out = lax.fori_loop(0, iters, it, carry, unroll=2)
    return out[0], out[4]


def _cg_into(N0_ref, N1_ref, v_ref, emb_ref, tabC_ref, tabP_ref):
    N0 = N0_ref[...]
    N1 = N1_ref[...]
    v8 = jnp.broadcast_to(v_ref[...], (8, D2))

    def nrhs(S, Nm):
        b = S - jnp.dot(S, Nm, preferred_element_type=jnp.float32)
        return b - jnp.dot(b, Nm, preferred_element_type=jnp.float32)

    xa, xb = _cg_rows2(N0, N1, nrhs(v8, N0), nrhs(v8, N1), _CG_IT1)
    e2 = xa[0:1, :]
    e3 = xb[0:1, :]

    rid = lax.broadcasted_iota(jnp.int32, (8, D2), 0)
    S = jnp.where(rid % 2 == 0, jnp.broadcast_to(e2, (8, D2)),
                  jnp.broadcast_to(e3, (8, D2)))
    x2 = _cg_rows(N0, nrhs(S, N0), _CG_IT2)

    p8 = jnp.where(rid < 2, v8,
         jnp.where(rid == 2, jnp.broadcast_to(e2, (8, D2)),
         jnp.where(rid == 3, jnp.broadcast_to(e3, (8, D2)),
         jnp.where(rid < 6, x2, jnp.zeros((8, D2), jnp.float32)))))
    tabC_ref[0:8, :] = p8
    tabC_ref[8:24, :] = emb_ref[...]
    tabP_ref[...] = p8


def _index_into(tt_ref, tv_ref, np_ref, g_ref):
    tt = tt_ref[...]
    tv = tv_ref[...]
    npos = np_ref[...]
    present = [jnp.sum(jnp.where(npos == v, 1, 0)) > 0 for v in range(6)]
    db = jnp.full((_IDX_R, _IDX_C), -1, jnp.int32)
    for v in range(5, -1, -1):
        db = jnp.where(jnp.logical_and(present[v], tv <= v), v, db)
    mp = jnp.int32(-1)
    for v in range(6):
        mp = jnp.where(present[v], jnp.int32(v), mp)
    db = jnp.where(db >= 0, db, mp)

    c = jnp.full((_IDX_R, _IDX_C), 19, jnp.int32)
    c = jnp.where(tt == 0, 8, c)
    c = jnp.where(tt == 1, 9 + tv, c)
    c = jnp.where(tt == 2, 13 + tv, c)
    c = jnp.where(tt == 4, db, c)
    g_ref[...] = c * 8 + npos


def _fused_body(tt_ref, tv_ref, np_ref, emb_ref, N0_ref, N1_ref, v_ref,
                g_ref, tabC_ref, tabP_ref):
    _cg_into(N0_ref, N1_ref, v_ref, emb_ref, tabC_ref, tabP_ref)
    _index_into(tt_ref, tv_ref, np_ref, g_ref)


def _fused_table_index(tt, tv, npos, emb16, N0, N1, v):
    return pl.pallas_call(
        _fused_body,
        out_shape=[
            jax.ShapeDtypeStruct((_IDX_R, _IDX_C), jnp.int32),
            jax.ShapeDtypeStruct((24, D2), jnp.float32),
            jax.ShapeDtypeStruct((8, D2), jnp.float32),
        ],
    )(tt, tv, npos, emb16, N0, N1, v)


def _assemble_body(tabC_hbm, tabP_hbm, idx_hbm, out_hbm,
                   tabC_v, tabP_v, g_v, buf0, buf1, sl0, sl1, sw0, sw1):
    sid = lax.axis_index("s")
    wid = sid * _NC + lax.axis_index("c")
    base = wid * _BPW

    @pl.when(sid == 0)
    def _():
        pltpu.sync_copy(tabC_hbm, tabC_v)
        pltpu.sync_copy(tabP_hbm, tabP_v)

    plsc.subcore_barrier()
    pltpu.sync_copy(idx_hbm.at[wid], g_v)
    bufs, sls, sws = (buf0, buf1), (sl0, sl1), (sw0, sw1)
    nch = _BPW // _KL

    def wait_w(b):
        pltpu.make_async_copy(bufs[b], out_hbm.at[pl.ds(base, _KL)],
                              sws[b]).wait()

    def drain_l(b):
        pltpu.make_async_copy(out_hbm.at[pl.ds(base, _KL)], bufs[b],
                              sls[b]).wait()

    def chunk(ck, b, first):
        if not first:
            wait_w(b)
        for h in range(_KL // 16):
            gvec = g_v[pl.ds(ck * _KL + h * 16, 16)]
            cvec = jax.lax.shift_right_logical(gvec, 3)
            pvec = jax.lax.bitwise_and(gvec, 7)
            for j in range(16):
                row = h * 16 + j
                pltpu.async_copy(tabC_v.at[cvec[j]],
                                 bufs[b].at[row, pl.ds(0, D2)], sls[b])
                pltpu.async_copy(tabP_v.at[pvec[j]],
                                 bufs[b].at[row, pl.ds(D2, D2)], sls[b])
        drain_l(b)
        pltpu.async_copy(bufs[b], out_hbm.at[pl.ds(base + ck * _KL, _KL)],
                         sws[b])

    chunk(0, 0, True)
    chunk(1, 1, True)

    def outer(t, carry):
        chunk(2 * t, 0, False)
        chunk(2 * t + 1, 1, False)
        return carry

    lax.fori_loop(1, nch // 2, outer, 0)
    wait_w(0)
    wait_w(1)


@functools.cache
def _assemble_rows_kernel():
    return functools.partial(
        pl.kernel,
        mesh=plsc.VectorSubcoreMesh(core_axis_name="c", subcore_axis_name="s"),
        out_type=jax.ShapeDtypeStruct((NTOK, DIM), jnp.float32),
        scratch_types=[
            pltpu.VMEM_SHARED((24, D2), jnp.float32),
            pltpu.VMEM_SHARED((8, D2), jnp.float32),
            pltpu.VMEM((_BPW,), jnp.int32),
            pltpu.VMEM((_KL, DIM), jnp.float32),
            pltpu.VMEM((_KL, DIM), jnp.float32),
            pltpu.SemaphoreType.DMA,
            pltpu.SemaphoreType.DMA,
            pltpu.SemaphoreType.DMA,
            pltpu.SemaphoreType.DMA,
        ],
    )(_assemble_body)


def _prepare(dense_batch, embeddings, primitives_raw, identity):
    f32 = jnp.float32
    X = jnp.tril(primitives_raw.astype(f32))
    A = X - jnp.swapaxes(X, -1, -2)
    N = 0.5 * A

    v = identity.astype(f32).reshape(1, D2)
    emb16 = jnp.pad(embeddings.astype(f32), ((0, 5), (0, 0)))
    tt = dense_batch[0].reshape(_IDX_R, _IDX_C)
    tv = dense_batch[1].reshape(_IDX_R, _IDX_C)
    npos = dense_batch[2].reshape(_IDX_R, _IDX_C)
    return tt, tv, npos, emb16, N[0], N[1], v


def kernel(dense_batch, embeddings, primitives_raw, identity):
    tt, tv, npos, emb16, N0, N1, v = _prepare(dense_batch, embeddings,
                                              primitives_raw, identity)
    g, tabC, tabP = _fused_table_index(tt, tv, npos, emb16, N0, N1, v)
    out = _assemble_rows_kernel()(tabC, tabP, g.reshape(_NW, _BPW))
    return out.reshape(4, 8192, DIM)
